# baseline (speedup 1.0000x reference)
import functools

import jax
import jax.numpy as jnp
from jax import lax
from jax.experimental import pallas as pl
from jax.experimental.pallas import tpu as pltpu
import jax.experimental.pallas.tpu_sc as plsc

N = 10000
E = 320000
D = 128

NC = 2
NS = 16

HALF = E // NC
PT2 = HALF // NS

ROWS_PER_TILE = N // NS


_TC_GRID = 10
_RB = N // _TC_GRID


def _tc_head_body(x_ref, w_ref, al_ref, ar_ref, feat_ref, el_ref, er_ref):
    f = jnp.dot(x_ref[...], w_ref[...], preferred_element_type=jnp.float32)
    feat_ref[...] = f
    el_ref[...] = jnp.sum(f * al_ref[...], axis=1, keepdims=True)
    er_ref[...] = jnp.sum(f * ar_ref[...], axis=1, keepdims=True)


def _tc_mid_body(pa_ref, pb_ref, dn0_ref, dn1_ref, b_ref, w_ref, al_ref,
                 ar_ref, feat_ref, el_ref, er_ref):
    dn = dn0_ref[...] + dn1_ref[...]
    inv = jnp.where(dn > 0.0, 1.0 / dn, 0.0)
    h = jnp.maximum((pa_ref[...] + pb_ref[...]) * inv + b_ref[...], 0.0)
    f = jnp.dot(h, w_ref[...], preferred_element_type=jnp.float32)
    feat_ref[...] = f
    el_ref[...] = jnp.sum(f * al_ref[...], axis=1, keepdims=True)
    er_ref[...] = jnp.sum(f * ar_ref[...], axis=1, keepdims=True)


def _tc_out_body(pa_ref, pb_ref, dn0_ref, dn1_ref, b_ref, o_ref):
    dn = dn0_ref[...] + dn1_ref[...]
    inv = jnp.where(dn > 0.0, 1.0 / dn, 0.0)
    o_ref[...] = jnp.maximum((pa_ref[...] + pb_ref[...]) * inv + b_ref[...],
                             0.0)


_row_spec = pl.BlockSpec((_RB, D), lambda i: (i, 0))
_col_spec = pl.BlockSpec((_RB, 1), lambda i: (i, 0))
_w_spec = pl.BlockSpec((D, D), lambda i: (0, 0))
_v_spec = pl.BlockSpec((1, D), lambda i: (0, 0))

_mat_out = jax.ShapeDtypeStruct((N, D), jnp.float32)
_colv_out = jax.ShapeDtypeStruct((N, 1), jnp.float32)

_tc_head = pl.pallas_call(
    _tc_head_body,
    grid=(_TC_GRID,),
    in_specs=[_row_spec, _w_spec, _v_spec, _v_spec],
    out_specs=[_row_spec, _col_spec, _col_spec],
    out_shape=[_mat_out, _colv_out, _colv_out],
)

_tc_mid = pl.pallas_call(
    _tc_mid_body,
    grid=(_TC_GRID,),
    in_specs=[_row_spec, _row_spec, _col_spec, _col_spec, _v_spec, _w_spec,
              _v_spec, _v_spec],
    out_specs=[_row_spec, _col_spec, _col_spec],
    out_shape=[_mat_out, _colv_out, _colv_out],
)

_tc_out = pl.pallas_call(
    _tc_out_body,
    grid=(_TC_GRID,),
    in_specs=[_row_spec, _row_spec, _col_spec, _col_spec, _v_spec],
    out_specs=_row_spec,
    out_shape=_mat_out,
)


_sc_mesh = plsc.VectorSubcoreMesh(
    core_axis_name="c", subcore_axis_name="s", num_cores=NC, num_subcores=NS)

BKA = 112
NBA = -(-PT2 // BKA)
JA = NBA - 1
DUPCA = (NBA * BKA - PT2) // 16
NCHA = BKA // 16


@functools.partial(
    pl.kernel,
    out_type=(
        jax.ShapeDtypeStruct((NC, N, D), jnp.float32),
        jax.ShapeDtypeStruct((NC * N,), jnp.float32),
    ),
    mesh=_sc_mesh,
    scratch_types=[
        pltpu.VMEM((BKA, D), jnp.float32),
        pltpu.VMEM((BKA, D), jnp.float32),
        pltpu.VMEM((BKA, D), jnp.float32),
        pltpu.VMEM((1, BKA), jnp.int32),
        pltpu.VMEM((1, BKA), jnp.int32),
        pltpu.VMEM((1, BKA), jnp.int32),
        pltpu.VMEM((1, BKA), jnp.int32),
        pltpu.VMEM((1, BKA), jnp.int32),
        pltpu.VMEM((1, BKA), jnp.int32),
        pltpu.VMEM((1, BKA), jnp.int32),
        pltpu.VMEM((1, BKA), jnp.int32),
        pltpu.VMEM((1, BKA), jnp.int32),
        pltpu.VMEM((BKA,), jnp.float32),
        pltpu.VMEM((BKA,), jnp.float32),
        pltpu.VMEM((BKA,), jnp.float32),
        pltpu.VMEM((BKA,), jnp.float32),
        pltpu.VMEM((BKA,), jnp.float32),
        pltpu.VMEM((BKA,), jnp.float32),
        pltpu.VMEM((BKA,), jnp.float32),
        pltpu.VMEM((BKA,), jnp.float32),
        pltpu.VMEM((BKA,), jnp.float32),
        pltpu.VMEM((640,), jnp.float32),
        pltpu.VMEM_SHARED((N,), jnp.float32),
        pltpu.VMEM_SHARED((N, D), jnp.float32),
        pltpu.SemaphoreType.DMA,
        pltpu.SemaphoreType.DMA,
        pltpu.SemaphoreType.DMA,
        pltpu.SemaphoreType.DMA,
        pltpu.SemaphoreType.DMA,
        pltpu.SemaphoreType.DMA,
        pltpu.SemaphoreType.DMA,
        pltpu.SemaphoreType.DMA,
        pltpu.SemaphoreType.DMA,
    ],
)
def _sc_edge(el_hbm, er_hbm, src_hbm, dst_hbm, feat_hbm,
             outp_hbm, denom_hbm,
             rows0, rows1, rows2, srcxA0, srcxA1, srcxA2,
             dstxA0, dstxA1, dstxA2, dstxS0, dstxS1, dstxS2,
             elA0, elA1, elA2, erA0, erA1, erA2, pmA0, pmA1, pmA2,
             zb, denom_s, out_s, semi0, semi1, semi2, semg0, semg1, semg2,
             semsc0, semsc1, semsc2):
    cid = lax.axis_index("c")
    sid = lax.axis_index("s")
    rows = (rows0, rows1, rows2)
    srcxA = (srcxA0, srcxA1, srcxA2)
    dstxA = (dstxA0, dstxA1, dstxA2)
    dstxS = (dstxS0, dstxS1, dstxS2)
    elA = (elA0, elA1, elA2)
    erA = (erA0, erA1, erA2)
    pmA = (pmA0, pmA1, pmA2)
    semi = (semi0, semi1, semi2)
    semg = (semg0, semg1, semg2)
    semsc = (semsc0, semsc1, semsc2)

    def zrow_body(t, _):
        i = t // 8
        k = t % 8
        rows0[i, pl.ds(k * 16, 16)] = jnp.zeros((16,), jnp.float32)
        return 0
    lax.fori_loop(0, BKA * 8, zrow_body, 0)

    def zp_body(k, _):
        zb[pl.ds(k * 16, 16)] = jnp.zeros((16,), jnp.float32)
        return 0
    lax.fori_loop(0, 40, zp_body, 0)

    r0 = (sid * ROWS_PER_TILE) // 8 * 8
    _WCH = ((0, 112), (112, 112), (224, 112), (336, 112), (448, 112),
            (560, 72))
    for off, ln in _WCH:
        pltpu.sync_copy(rows0.at[pl.ds(0, ln)],
                        out_s.at[pl.ds(r0 + off, ln)])
    r0d = (sid * ROWS_PER_TILE) // 16 * 16
    pltpu.sync_copy(zb, denom_s.at[pl.ds(r0d, 640)])

    plsc.subcore_barrier()

    def baseA(j):
        return jnp.minimum(j * BKA, PT2 - BKA)

    def issue_idx_A(b, j):
        ghb = cid * HALF + sid * PT2 + baseA(j)
        pltpu.async_copy(src_hbm.at[pl.ds(ghb, BKA)], srcxA[b].at[0],
                         semi[b])
        pltpu.async_copy(dst_hbm.at[pl.ds(ghb, BKA)], dstxA[b].at[0],
                         semi[b])

    def wait_idx_A(b):
        pltpu.make_async_copy(src_hbm.at[pl.ds(0, BKA)], srcxA[b].at[0],
                              semi[b]).wait()
        pltpu.make_async_copy(src_hbm.at[pl.ds(0, BKA)], dstxA[b].at[0],
                              semi[b]).wait()

    def issue_g_A(b):
        hf = BKA // 2
        pltpu.async_copy(feat_hbm.at[srcxA[b].at[0, pl.ds(0, hf)]],
                         rows[b].at[pl.ds(0, hf)], semg[b])
        pltpu.async_copy(feat_hbm.at[srcxA[b].at[0, pl.ds(hf, hf)]],
                         rows[b].at[pl.ds(hf, hf)], semg[b])
        pltpu.async_copy(el_hbm.at[srcxA[b].at[0]], elA[b], semg[b])
        pltpu.async_copy(er_hbm.at[dstxA[b].at[0]], erA[b], semg[b])

    def wait_g_A(b):
        pltpu.make_async_copy(feat_hbm.at[pl.ds(0, BKA // 2)],
                              rows[b].at[pl.ds(0, BKA // 2)],
                              semg[b]).wait()
        pltpu.make_async_copy(feat_hbm.at[pl.ds(0, BKA // 2)],
                              rows[b].at[pl.ds(0, BKA // 2)],
                              semg[b]).wait()
        pltpu.make_async_copy(el_hbm.at[pl.ds(0, BKA)], elA[b],
                              semg[b]).wait()
        pltpu.make_async_copy(el_hbm.at[pl.ds(0, BKA)], erA[b],
                              semg[b]).wait()

    def wait_scat_A(b):
        pltpu.make_async_copy(feat_hbm.at[pl.ds(0, BKA)], rows[b],
                              semsc[b]).wait()
        pltpu.make_async_copy(el_hbm.at[pl.ds(0, BKA)], pmA[b],
                              semsc[b]).wait()

    def bodyA(b, j):
        bn = (b + 1) % 3
        bn2 = (b + 2) % 3

        @pl.when(j >= 2)
        def _():
            wait_scat_A(bn)

        @pl.when(j + 1 <= JA)
        def _():
            wait_idx_A(bn)
            issue_g_A(bn)
        wait_g_A(b)
        if True:
            is_tail = j == JA
            for k in range(NCHA):
                s = elA[b][pl.ds(k * 16, 16)] + erA[b][pl.ds(k * 16, 16)]
                p = jnp.exp(jnp.maximum(s, 0.2 * s))
                if k < DUPCA:
                    p = jnp.where(is_tail, 0.0, p)
                pmA[b][pl.ds(k * 16, 16)] = p
                dstxS[b][0, pl.ds(k * 16, 16)] = dstxA[b][0, pl.ds(k * 16,
                                                                   16)]

            def g_body(g, _):
                a16 = pmA[b][pl.ds(g * 16, 16)]
                for l in range(16):
                    ab = jnp.broadcast_to(a16[l], (16,))
                    e = g * 16 + l
                    for c8 in range(8):
                        sl = pl.ds(c8 * 16, 16)
                        rows[b][e, sl] = rows[b][e, sl] * ab
                return 0

            lax.fori_loop(0, NCHA, g_body, 0)
            pltpu.async_copy(rows[b], out_s.at[dstxS[b].at[0]],
                             semsc[b], add=True)
            pltpu.async_copy(pmA[b], denom_s.at[dstxS[b].at[0]],
                             semsc[b], add=True)

            @pl.when(j + 2 <= JA)
            def _():
                issue_idx_A(bn2, j + 2)

    issue_idx_A(0, 0)
    wait_idx_A(0)
    issue_g_A(0)
    issue_idx_A(1, 1)

    def tripleA(i, _):
        bodyA(0, 3 * i)
        bodyA(1, 3 * i + 1)
        bodyA(2, 3 * i + 2)
        return 0

    lax.fori_loop(0, NBA // 3, tripleA, 0)
    wait_scat_A((JA - 1) % 3)
    wait_scat_A(JA % 3)

    plsc.subcore_barrier()

    for off, ln in _WCH:
        pltpu.sync_copy(out_s.at[pl.ds(r0 + off, ln)],
                        rows0.at[pl.ds(0, ln)])
        pltpu.sync_copy(rows0.at[pl.ds(0, ln)],
                        outp_hbm.at[cid, pl.ds(r0 + off, ln)])

    pltpu.sync_copy(denom_s.at[pl.ds(r0d, 640)], zb)
    pltpu.sync_copy(zb, denom_hbm.at[pl.ds(cid * N + r0d, 640)])


def kernel(x, edge_index, W1, attn_l1, attn_r1, b1, W2, attn_l2, attn_r2, b2):
    src = edge_index[0].astype(jnp.int32)
    dst = edge_index[1].astype(jnp.int32)

    feat1, el1, er1 = _tc_head(x, W1, attn_l1.reshape(1, D),
                               attn_r1.reshape(1, D))
    outp1, dn1 = _sc_edge(el1.reshape(N), er1.reshape(N), src, dst, feat1)
    feat2, el2, er2 = _tc_mid(outp1[0], outp1[1], dn1[:N].reshape(N, 1),
                              dn1[N:].reshape(N, 1), b1.reshape(1, D), W2,
                              attn_l2.reshape(1, D), attn_r2.reshape(1, D))
    outp2, dn2 = _sc_edge(el2.reshape(N), er2.reshape(N), src, dst, feat2)
    return _tc_out(outp2[0], outp2[1], dn2[:N].reshape(N, 1),
                   dn2[N:].reshape(N, 1), b2.reshape(1, D))


# speedup vs baseline: 1.0191x; 1.0191x over previous
import functools

import jax
import jax.numpy as jnp
from jax import lax
from jax.experimental import pallas as pl
from jax.experimental.pallas import tpu as pltpu
import jax.experimental.pallas.tpu_sc as plsc

N = 10000
E = 320000
D = 128

NC = 2
NS = 16

HALF = E // NC
PT2 = HALF // NS

ROWS_PER_TILE = N // NS


_TC_GRID = 5
_RB = N // _TC_GRID


def _tc_head_body(x_ref, w_ref, al_ref, ar_ref, feat_ref, el_ref, er_ref):
    f = jnp.dot(x_ref[...], w_ref[...], preferred_element_type=jnp.float32)
    feat_ref[...] = f
    el_ref[...] = jnp.sum(f * al_ref[...], axis=1, keepdims=True)
    er_ref[...] = jnp.sum(f * ar_ref[...], axis=1, keepdims=True)


def _tc_mid_body(pa_ref, pb_ref, dn0_ref, dn1_ref, b_ref, w_ref, al_ref,
                 ar_ref, feat_ref, el_ref, er_ref):
    dn = dn0_ref[...] + dn1_ref[...]
    inv = jnp.where(dn > 0.0, 1.0 / dn, 0.0)
    h = jnp.maximum((pa_ref[...] + pb_ref[...]) * inv + b_ref[...], 0.0)
    f = jnp.dot(h, w_ref[...], preferred_element_type=jnp.float32)
    feat_ref[...] = f
    el_ref[...] = jnp.sum(f * al_ref[...], axis=1, keepdims=True)
    er_ref[...] = jnp.sum(f * ar_ref[...], axis=1, keepdims=True)


def _tc_out_body(pa_ref, pb_ref, dn0_ref, dn1_ref, b_ref, o_ref):
    dn = dn0_ref[...] + dn1_ref[...]
    inv = jnp.where(dn > 0.0, 1.0 / dn, 0.0)
    o_ref[...] = jnp.maximum((pa_ref[...] + pb_ref[...]) * inv + b_ref[...],
                             0.0)


_row_spec = pl.BlockSpec((_RB, D), lambda i: (i, 0))
_col_spec = pl.BlockSpec((_RB, 1), lambda i: (i, 0))
_w_spec = pl.BlockSpec((D, D), lambda i: (0, 0))
_v_spec = pl.BlockSpec((1, D), lambda i: (0, 0))

_mat_out = jax.ShapeDtypeStruct((N, D), jnp.float32)
_colv_out = jax.ShapeDtypeStruct((N, 1), jnp.float32)

_tc_head = pl.pallas_call(
    _tc_head_body,
    grid=(_TC_GRID,),
    in_specs=[_row_spec, _w_spec, _v_spec, _v_spec],
    out_specs=[_row_spec, _col_spec, _col_spec],
    out_shape=[_mat_out, _colv_out, _colv_out],
)

_tc_mid = pl.pallas_call(
    _tc_mid_body,
    grid=(_TC_GRID,),
    in_specs=[_row_spec, _row_spec, _col_spec, _col_spec, _v_spec, _w_spec,
              _v_spec, _v_spec],
    out_specs=[_row_spec, _col_spec, _col_spec],
    out_shape=[_mat_out, _colv_out, _colv_out],
)

_tc_out = pl.pallas_call(
    _tc_out_body,
    grid=(_TC_GRID,),
    in_specs=[_row_spec, _row_spec, _col_spec, _col_spec, _v_spec],
    out_specs=_row_spec,
    out_shape=_mat_out,
)


_sc_mesh = plsc.VectorSubcoreMesh(
    core_axis_name="c", subcore_axis_name="s", num_cores=NC, num_subcores=NS)

BKA = 112
NBA = -(-PT2 // BKA)
JA = NBA - 1
DUPCA = (NBA * BKA - PT2) // 16
NCHA = BKA // 16


@functools.partial(
    pl.kernel,
    out_type=(
        jax.ShapeDtypeStruct((NC, N, D), jnp.float32),
        jax.ShapeDtypeStruct((NC * N,), jnp.float32),
    ),
    mesh=_sc_mesh,
    scratch_types=[
        pltpu.VMEM((BKA, D), jnp.float32),
        pltpu.VMEM((BKA, D), jnp.float32),
        pltpu.VMEM((BKA, D), jnp.float32),
        pltpu.VMEM((1, BKA), jnp.int32),
        pltpu.VMEM((1, BKA), jnp.int32),
        pltpu.VMEM((1, BKA), jnp.int32),
        pltpu.VMEM((1, BKA), jnp.int32),
        pltpu.VMEM((1, BKA), jnp.int32),
        pltpu.VMEM((1, BKA), jnp.int32),
        pltpu.VMEM((1, BKA), jnp.int32),
        pltpu.VMEM((1, BKA), jnp.int32),
        pltpu.VMEM((1, BKA), jnp.int32),
        pltpu.VMEM((BKA,), jnp.float32),
        pltpu.VMEM((BKA,), jnp.float32),
        pltpu.VMEM((BKA,), jnp.float32),
        pltpu.VMEM((BKA,), jnp.float32),
        pltpu.VMEM((BKA,), jnp.float32),
        pltpu.VMEM((BKA,), jnp.float32),
        pltpu.VMEM((BKA,), jnp.float32),
        pltpu.VMEM((BKA,), jnp.float32),
        pltpu.VMEM((BKA,), jnp.float32),
        pltpu.VMEM((640,), jnp.float32),
        pltpu.VMEM_SHARED((N,), jnp.float32),
        pltpu.VMEM_SHARED((N, D), jnp.float32),
        pltpu.SemaphoreType.DMA,
        pltpu.SemaphoreType.DMA,
        pltpu.SemaphoreType.DMA,
        pltpu.SemaphoreType.DMA,
        pltpu.SemaphoreType.DMA,
        pltpu.SemaphoreType.DMA,
        pltpu.SemaphoreType.DMA,
        pltpu.SemaphoreType.DMA,
        pltpu.SemaphoreType.DMA,
    ],
)
def _sc_edge(el_hbm, er_hbm, src_hbm, dst_hbm, feat_hbm,
             outp_hbm, denom_hbm,
             rows0, rows1, rows2, srcxA0, srcxA1, srcxA2,
             dstxA0, dstxA1, dstxA2, dstxS0, dstxS1, dstxS2,
             elA0, elA1, elA2, erA0, erA1, erA2, pmA0, pmA1, pmA2,
             zb, denom_s, out_s, semi0, semi1, semi2, semg0, semg1, semg2,
             semsc0, semsc1, semsc2):
    cid = lax.axis_index("c")
    sid = lax.axis_index("s")
    rows = (rows0, rows1, rows2)
    srcxA = (srcxA0, srcxA1, srcxA2)
    dstxA = (dstxA0, dstxA1, dstxA2)
    dstxS = (dstxS0, dstxS1, dstxS2)
    elA = (elA0, elA1, elA2)
    erA = (erA0, erA1, erA2)
    pmA = (pmA0, pmA1, pmA2)
    semi = (semi0, semi1, semi2)
    semg = (semg0, semg1, semg2)
    semsc = (semsc0, semsc1, semsc2)

    def zrow_body(t, _):
        i = t // 8
        k = t % 8
        rows0[i, pl.ds(k * 16, 16)] = jnp.zeros((16,), jnp.float32)
        return 0
    lax.fori_loop(0, BKA * 8, zrow_body, 0)

    def zp_body(k, _):
        zb[pl.ds(k * 16, 16)] = jnp.zeros((16,), jnp.float32)
        return 0
    lax.fori_loop(0, 40, zp_body, 0)

    r0 = (sid * ROWS_PER_TILE) // 8 * 8
    _WCH = ((0, 112), (112, 112), (224, 112), (336, 112), (448, 112),
            (560, 72))
    for off, ln in _WCH:
        pltpu.sync_copy(rows0.at[pl.ds(0, ln)],
                        out_s.at[pl.ds(r0 + off, ln)])
    r0d = (sid * ROWS_PER_TILE) // 16 * 16
    pltpu.sync_copy(zb, denom_s.at[pl.ds(r0d, 640)])

    plsc.subcore_barrier()

    def baseA(j):
        return jnp.minimum(j * BKA, PT2 - BKA)

    def issue_idx_A(b, j):
        ghb = cid * HALF + sid * PT2 + baseA(j)
        pltpu.async_copy(src_hbm.at[pl.ds(ghb, BKA)], srcxA[b].at[0],
                         semi[b])
        pltpu.async_copy(dst_hbm.at[pl.ds(ghb, BKA)], dstxA[b].at[0],
                         semi[b])

    def wait_idx_A(b):
        pltpu.make_async_copy(src_hbm.at[pl.ds(0, BKA)], srcxA[b].at[0],
                              semi[b]).wait()
        pltpu.make_async_copy(src_hbm.at[pl.ds(0, BKA)], dstxA[b].at[0],
                              semi[b]).wait()

    def issue_g_A(b):
        pltpu.async_copy(feat_hbm.at[srcxA[b].at[0]], rows[b], semg[b])
        pltpu.async_copy(el_hbm.at[srcxA[b].at[0]], elA[b], semg[b])
        pltpu.async_copy(er_hbm.at[dstxA[b].at[0]], erA[b], semg[b])

    def wait_g_A(b):
        pltpu.make_async_copy(feat_hbm.at[pl.ds(0, BKA)], rows[b],
                              semg[b]).wait()
        pltpu.make_async_copy(el_hbm.at[pl.ds(0, BKA)], elA[b],
                              semg[b]).wait()
        pltpu.make_async_copy(el_hbm.at[pl.ds(0, BKA)], erA[b],
                              semg[b]).wait()

    def wait_scat_A(b):
        pltpu.make_async_copy(feat_hbm.at[pl.ds(0, BKA)], rows[b],
                              semsc[b]).wait()
        pltpu.make_async_copy(el_hbm.at[pl.ds(0, BKA)], pmA[b],
                              semsc[b]).wait()

    def bodyA(b, j):
        bn = (b + 1) % 3
        bn2 = (b + 2) % 3

        @pl.when(j >= 2)
        def _():
            wait_scat_A(bn)

        @pl.when(j + 1 <= JA)
        def _():
            wait_idx_A(bn)
            issue_g_A(bn)
        wait_g_A(b)
        if True:
            is_tail = j == JA
            for k in range(NCHA):
                s = elA[b][pl.ds(k * 16, 16)] + erA[b][pl.ds(k * 16, 16)]
                p = jnp.exp(jnp.maximum(s, 0.2 * s))
                if k < DUPCA:
                    p = jnp.where(is_tail, 0.0, p)
                pmA[b][pl.ds(k * 16, 16)] = p
                dstxS[b][0, pl.ds(k * 16, 16)] = dstxA[b][0, pl.ds(k * 16,
                                                                   16)]

            def g_body(g, _):
                a16 = pmA[b][pl.ds(g * 16, 16)]
                for l in range(16):
                    ab = jnp.broadcast_to(a16[l], (16,))
                    e = g * 16 + l
                    for c8 in range(8):
                        sl = pl.ds(c8 * 16, 16)
                        rows[b][e, sl] = rows[b][e, sl] * ab
                return 0

            lax.fori_loop(0, NCHA, g_body, 0)
            pltpu.async_copy(rows[b], out_s.at[dstxS[b].at[0]],
                             semsc[b], add=True)
            pltpu.async_copy(pmA[b], denom_s.at[dstxS[b].at[0]],
                             semsc[b], add=True)

            @pl.when(j + 2 <= JA)
            def _():
                issue_idx_A(bn2, j + 2)

    issue_idx_A(0, 0)
    wait_idx_A(0)
    issue_g_A(0)
    issue_idx_A(1, 1)

    def tripleA(i, _):
        bodyA(0, 3 * i)
        bodyA(1, 3 * i + 1)
        bodyA(2, 3 * i + 2)
        return 0

    lax.fori_loop(0, NBA // 3, tripleA, 0)
    wait_scat_A((JA - 1) % 3)
    wait_scat_A(JA % 3)

    plsc.subcore_barrier()

    for off, ln in _WCH:
        pltpu.sync_copy(out_s.at[pl.ds(r0 + off, ln)],
                        rows0.at[pl.ds(0, ln)])
        pltpu.sync_copy(rows0.at[pl.ds(0, ln)],
                        outp_hbm.at[cid, pl.ds(r0 + off, ln)])

    pltpu.sync_copy(denom_s.at[pl.ds(r0d, 640)], zb)
    pltpu.sync_copy(zb, denom_hbm.at[pl.ds(cid * N + r0d, 640)])


def kernel(x, edge_index, W1, attn_l1, attn_r1, b1, W2, attn_l2, attn_r2, b2):
    src = edge_index[0].astype(jnp.int32)
    dst = edge_index[1].astype(jnp.int32)

    feat1, el1, er1 = _tc_head(x, W1, attn_l1.reshape(1, D),
                               attn_r1.reshape(1, D))
    outp1, dn1 = _sc_edge(el1.reshape(N), er1.reshape(N), src, dst, feat1)
    feat2, el2, er2 = _tc_mid(outp1[0], outp1[1], dn1[:N].reshape(N, 1),
                              dn1[N:].reshape(N, 1), b1.reshape(1, D), W2,
                              attn_l2.reshape(1, D), attn_r2.reshape(1, D))
    outp2, dn2 = _sc_edge(el2.reshape(N), er2.reshape(N), src, dst, feat2)
    return _tc_out(outp2[0], outp2[1], dn2[:N].reshape(N, 1),
                   dn2[N:].reshape(N, 1), b2.reshape(1, D))
